# baseline (speedup 1.0000x reference)
import functools

import jax
import jax.numpy as jnp
from jax import lax
from jax.experimental import pallas as pl
from jax.experimental.pallas import tpu as pltpu
from jax.experimental.pallas import tpu_sc as plsc

DIM = 64
N_CTX = 200
N_NEG = 12800
N_REAL = N_CTX + N_NEG
NC, NS = 2, 16
NW = NC * NS
BPW = 416
B = NW * BPW
NCHUNK, CH = 4, 104

_mesh = plsc.VectorSubcoreMesh(core_axis_name="c", subcore_axis_name="s")


def _sc_gather_body(out_emb, in_emb, idx_hbm, tidx_hbm, g_out, t_out,
                    idx_v, rows_v, tidx_v, trow_v, sem):
    wid = lax.axis_index("s") * NC + lax.axis_index("c")
    base = wid * BPW
    pltpu.sync_copy(idx_hbm.at[wid], idx_v)
    descs = [
        pltpu.async_copy(out_emb.at[idx_v.at[k]],
                         rows_v.at[pl.ds(k * CH, CH)], sem)
        for k in range(NCHUNK)
    ]
    for d in descs:
        d.wait()
    pltpu.sync_copy(rows_v, g_out.at[pl.ds(base, BPW)])

    @pl.when(wid == 0)
    def _():
        pltpu.sync_copy(tidx_hbm, tidx_v)
        pltpu.async_copy(in_emb.at[tidx_v], trow_v, sem).wait()
        pltpu.sync_copy(trow_v, t_out)


_sc_gather = functools.partial(
    pl.kernel,
    mesh=_mesh,
    compiler_params=pltpu.CompilerParams(use_tc_tiling_on_sc=False),
    out_type=(
        jax.ShapeDtypeStruct((B, DIM), jnp.float32),
        jax.ShapeDtypeStruct((8, DIM), jnp.float32),
    ),
    scratch_types=[
        pltpu.VMEM((NCHUNK, CH), jnp.int32),
        pltpu.VMEM((BPW, DIM), jnp.float32),
        pltpu.VMEM((8,), jnp.int32),
        pltpu.VMEM((8, DIM), jnp.float32),
        pltpu.SemaphoreType.DMA,
    ],
)(_sc_gather_body)


def _tc_body(g_ref, t_ref, sign_ref, out_ref):
    t_row = t_ref[0:1, :]
    g = g_ref[...]
    s = lax.dot_general(t_row, g, (((1,), (1,)), ((), ())),
                        preferred_element_type=jnp.float32)
    sign = sign_ref[...]
    z = s * sign
    ls = jnp.minimum(z, 0.0) - jnp.log(1.0 + jnp.exp(-jnp.abs(z)))
    contrib = jnp.where(sign != 0.0, ls, 0.0)
    out_ref[0, 0] = -jnp.sum(contrib)


def kernel(input_embeddings, output_embeddings, target, context,
           negative_samples):
    idx = jnp.concatenate([
        context.astype(jnp.int32),
        negative_samples.reshape(-1).astype(jnp.int32),
        jnp.zeros((B - N_REAL,), jnp.int32),
    ]).reshape(NW, NCHUNK, CH)
    sign = jnp.concatenate([
        jnp.ones((N_CTX,), jnp.float32),
        jnp.full((N_NEG,), -1.0, jnp.float32),
        jnp.zeros((B - N_REAL,), jnp.float32),
    ]).reshape(1, B)
    tidx = jnp.full((8,), target, jnp.int32)

    gathered, t8 = _sc_gather(output_embeddings, input_embeddings, idx, tidx)

    loss = pl.pallas_call(
        _tc_body,
        out_shape=jax.ShapeDtypeStruct((1, 1), jnp.float32),
        out_specs=pl.BlockSpec(memory_space=pltpu.SMEM),
    )(gathered, t8, sign)
    return loss.reshape(())


# speedup vs baseline: 9.3714x; 9.3714x over previous
import functools

import jax
import jax.numpy as jnp
from jax import lax
from jax.experimental import pallas as pl
from jax.experimental.pallas import tpu as pltpu
from jax.experimental.pallas import tpu_sc as plsc

V = 1_000_000
DIM = 64
N_CTX = 200
N_NEG = 12800
N_REAL = N_CTX + N_NEG
NC, NS = 2, 16
NW = NC * NS
IPW = 512
BTOT = NW * IPW
NCH = IPW // 128
NB = 32768
GRID = (V + NB - 1) // NB


def _mv_body(scal_ref, in_t_ref, out_t_ref, y_ref):
    tmod = scal_ref[1]
    onehot = (lax.broadcasted_iota(jnp.int32, (128, 1), 0) == tmod)
    t_col = lax.dot_general(in_t_ref[...], onehot.astype(jnp.float32),
                            (((1,), (0,)), ((), ())),
                            preferred_element_type=jnp.float32)
    s = lax.dot_general(t_col, out_t_ref[...], (((0,), (0,)), ((), ())),
                        preferred_element_type=jnp.float32)
    y_ref[...] = s.reshape((NB,))


def _matvec(in_t, out_t, scal):
    grid_spec = pltpu.PrefetchScalarGridSpec(
        num_scalar_prefetch=1,
        grid=(GRID,),
        in_specs=[
            pl.BlockSpec((DIM, 128), lambda i, s: (0, s[0])),
            pl.BlockSpec((DIM, NB), lambda i, s: (0, i)),
        ],
        out_specs=pl.BlockSpec((NB,), lambda i, s: (i,)),
    )
    return pl.pallas_call(
        _mv_body,
        grid_spec=grid_spec,
        out_shape=jax.ShapeDtypeStruct((GRID * NB,), jnp.float32),
    )(scal, in_t, out_t)


def _sc_body(y_hbm, idx_hbm, sign_hbm, out_hbm, idx_v, sign_v, g_v, acc_v,
             sem):
    wid = lax.axis_index("s") * NC + lax.axis_index("c")
    pltpu.sync_copy(idx_hbm.at[wid], idx_v)
    pltpu.sync_copy(sign_hbm.at[wid], sign_v)
    descs = [
        pltpu.async_copy(y_hbm.at[idx_v.at[k]],
                         g_v.at[pl.ds(k * 128, 128)], sem)
        for k in range(NCH)
    ]
    for dsc in descs:
        dsc.wait()

    def cbody(c, tot):
        z = g_v[pl.ds(c * 16, 16)] * sign_v[pl.ds(c * 16, 16)]
        sgn = sign_v[pl.ds(c * 16, 16)]
        u = jnp.exp(-jnp.abs(z))
        y = u / (2.0 + u)
        y2 = y * y
        l1p = y * (2.0 + y2 * (2.0 / 3.0 + y2 * (2.0 / 5.0 + y2 * (
            2.0 / 7.0 + y2 * (2.0 / 9.0 + y2 * (2.0 / 11.0))))))
        contrib = jnp.minimum(z, 0.0) - l1p
        contrib = jnp.where(sgn == 0.0, 0.0, contrib)
        return tot + contrib

    tot = lax.fori_loop(0, IPW // 16, cbody, jnp.zeros((16,), jnp.float32))
    acc_v[...] = tot
    pltpu.sync_copy(acc_v, out_hbm.at[pl.ds(wid * 16, 16)])


_sc_reduce = functools.partial(
    pl.kernel,
    mesh=plsc.VectorSubcoreMesh(core_axis_name="c", subcore_axis_name="s"),
    compiler_params=pltpu.CompilerParams(use_tc_tiling_on_sc=False),
    out_type=jax.ShapeDtypeStruct((NW * 16,), jnp.float32),
    scratch_types=[
        pltpu.VMEM((NCH, 128), jnp.int32),
        pltpu.VMEM((IPW,), jnp.float32),
        pltpu.VMEM((IPW,), jnp.float32),
        pltpu.VMEM((16,), jnp.float32),
        pltpu.SemaphoreType.DMA,
    ],
)(_sc_body)


def kernel(input_embeddings, output_embeddings, target, context,
           negative_samples):
    tgt = jnp.asarray(target, jnp.int32)
    scal = jnp.stack([tgt // 128, tgt % 128])
    y = _matvec(input_embeddings.T, output_embeddings.T, scal)

    idx = jnp.concatenate([
        context.astype(jnp.int32),
        negative_samples.reshape(-1).astype(jnp.int32),
        jnp.zeros((BTOT - N_REAL,), jnp.int32),
    ]).reshape(NW, NCH, 128)
    sign = jnp.concatenate([
        jnp.ones((N_CTX,), jnp.float32),
        jnp.full((N_NEG,), -1.0, jnp.float32),
        jnp.zeros((BTOT - N_REAL,), jnp.float32),
    ]).reshape(NW, IPW)

    partials = _sc_reduce(y, idx, sign)
    return -jnp.sum(partials)
